# Initial kernel scaffold; baseline (speedup 1.0000x reference)
import jax
import jax.numpy as jnp
from jax.experimental import pallas as pl


def kernel(token_ids, embedding_matrix):
    raise NotImplementedError("write your pallas kernel here")


# speedup vs baseline: 1.3094x; 1.3094x over previous
import jax
import jax.numpy as jnp
from jax import lax
from jax.experimental import pallas as pl
from jax.experimental.pallas import tpu as pltpu
from jax.experimental.pallas import tpu_sc as plsc

D = 32
NC, NS = 2, 16
NW = NC * NS
CHUNK = 128
K = 4
B = 16384 * 50
BPW = B // NW
NSTEPS = BPW // CHUNK
NROUNDS = NSTEPS // K


def _emb_body(idx_hbm, table_hbm, out_hbm, idx_v, *scr):
    rows = [[scr[g * K + t] for t in range(K)] for g in range(2)]
    gsem = [scr[2 * K], scr[2 * K + 1]]
    wsem = [scr[2 * K + 2], scr[2 * K + 3]]

    wid = lax.axis_index("s") * NC + lax.axis_index("c")
    pltpu.sync_copy(idx_hbm.at[wid], idx_v)

    def g_copy(g, t, r):
        step = r * K + t
        return pltpu.make_async_copy(
            table_hbm.at[idx_v.at[step]], rows[g][t], gsem[g])

    def w_copy(g, t, r):
        step = r * K + t
        return pltpu.make_async_copy(
            rows[g][t], out_hbm.at[wid, step], wsem[g])

    def fire_g(g, r):
        for t in range(K):
            g_copy(g, t, r).start()

    def drain_g(g, r):
        for t in range(K):
            g_copy(g, t, r).wait()

    def fire_w(g, r):
        for t in range(K):
            w_copy(g, t, r).start()

    def drain_w(g, r):
        for t in range(K):
            w_copy(g, t, r).wait()

    fire_g(0, 0)

    def process(g, r):
        @pl.when(r >= 1)
        def _():
            drain_w(1 - g, r - 1)

        @pl.when(r + 1 < NROUNDS)
        def _():
            fire_g(1 - g, r + 1)

        drain_g(g, r)
        fire_w(g, r)

    def body(i, carry):
        process(0, 2 * i)
        process(1, 2 * i + 1)
        return carry

    lax.fori_loop(0, NROUNDS // 2, body, 0)
    drain_w((NROUNDS - 1) % 2, NROUNDS - 1)


def kernel(token_ids, embedding_matrix):
    idx = token_ids.reshape(NW, NSTEPS, CHUNK).astype(jnp.int32)
    call = pl.kernel(
        _emb_body,
        out_type=jax.ShapeDtypeStruct((NW, NSTEPS, CHUNK, D), jnp.float32),
        mesh=plsc.VectorSubcoreMesh(core_axis_name="c", subcore_axis_name="s"),
        compiler_params=pltpu.CompilerParams(use_tc_tiling_on_sc=False),
        scratch_types=(
            [pltpu.VMEM((NSTEPS, CHUNK), jnp.int32)]
            + [pltpu.VMEM((CHUNK, D), jnp.float32) for _ in range(2 * K)]
            + [pltpu.SemaphoreType.DMA] * 4
        ),
    )
    out = call(idx, embedding_matrix)
    return out.reshape(token_ids.shape + (D,))
